# Initial kernel scaffold; baseline (speedup 1.0000x reference)
import jax
import jax.numpy as jnp
from jax.experimental import pallas as pl


def kernel(x, edge_index, batch, W1, as1, ad1, b1, W2, as2, ad2, b2, Wf, bf):
    raise NotImplementedError("write your pallas kernel here")


# speedup vs baseline: 33.5401x; 33.5401x over previous
import functools

import jax
import jax.numpy as jnp
from jax import lax
from jax.experimental import pallas as pl
from jax.experimental.pallas import tpu as pltpu
from jax.experimental.pallas import tpu_sc as plsc

N = 10000
NPAD = 10240
D_IN = 128
HID = 64
NUM_CLASSES = 10
NUM_GRAPHS = 128
E = 320000
E2 = E + N
NC, NS, L = 2, 16, 16
NW = NC * NS
KB = 128
BPW = 81
E2PAD = NW * BPW * KB
EROWS = E2PAD // KB
PAD_NODE = 10016
ROWS_PER_TILE = NPAD // NS


def _tc_mm_body(x_ref, w_ref, a_ref, h_ref, aux_ref):
    h = jnp.dot(x_ref[...], w_ref[...], preferred_element_type=jnp.float32)
    h_ref[...] = h
    aux_ref[...] = jnp.dot(h, a_ref[...], preferred_element_type=jnp.float32)


def _tc_mm(x, w, a):
    m, k = x.shape
    n = w.shape[1]
    blk = 1024
    return pl.pallas_call(
        _tc_mm_body,
        grid=(m // blk,),
        in_specs=[
            pl.BlockSpec((blk, k), lambda i: (i, 0)),
            pl.BlockSpec((k, n), lambda i: (0, 0)),
            pl.BlockSpec((n, 128), lambda i: (0, 0)),
        ],
        out_specs=[
            pl.BlockSpec((blk, n), lambda i: (i, 0)),
            pl.BlockSpec((blk, 128), lambda i: (i, 0)),
        ],
        out_shape=[
            jax.ShapeDtypeStruct((m, n), jnp.float32),
            jax.ShapeDtypeStruct((m, 128), jnp.float32),
        ],
    )(x, w, a)


def _tc_norm_mm_body(p_ref, sp_ref, b_ref, w_ref, a_ref, h_ref, aux_ref):
    acc = p_ref[0] + p_ref[1]
    s = jnp.sum(sp_ref[...], axis=0)
    o = acc / (s[:, None] + 1e-16) + b_ref[...]
    o = jnp.maximum(o, 0.0)
    h = jnp.dot(o, w_ref[...], preferred_element_type=jnp.float32)
    h_ref[...] = h
    aux_ref[...] = jnp.dot(h, a_ref[...], preferred_element_type=jnp.float32)


def _tc_norm_mm(partials, sparts, b, w, a):
    blk = 1024
    n = w.shape[1]
    return pl.pallas_call(
        _tc_norm_mm_body,
        grid=(NPAD // blk,),
        in_specs=[
            pl.BlockSpec((2, blk, HID), lambda i: (0, i, 0)),
            pl.BlockSpec((NW, blk), lambda i: (0, i)),
            pl.BlockSpec((1, HID), lambda i: (0, 0)),
            pl.BlockSpec((HID, n), lambda i: (0, 0)),
            pl.BlockSpec((n, 128), lambda i: (0, 0)),
        ],
        out_specs=[
            pl.BlockSpec((blk, n), lambda i: (i, 0)),
            pl.BlockSpec((blk, 128), lambda i: (i, 0)),
        ],
        out_shape=[
            jax.ShapeDtypeStruct((NPAD, n), jnp.float32),
            jax.ShapeDtypeStruct((NPAD, 128), jnp.float32),
        ],
    )(partials, sparts, b, w, a)


def _tc_pool_body(p_ref, sp_ref, b_ref, bt_ref, wf_ref, bf_ref, out_ref,
                  pooled_acc):
    i = pl.program_id(0)

    @pl.when(i == 0)
    def _():
        pooled_acc[...] = jnp.zeros_like(pooled_acc)

    acc = p_ref[0] + p_ref[1]
    s = jnp.sum(sp_ref[...], axis=0)
    o = acc / (s[:, None] + 1e-16) + b_ref[...]
    o = jnp.maximum(o, 0.0)
    bv = bt_ref[...]
    col = lax.broadcasted_iota(jnp.int32, (NUM_GRAPHS, bv.shape[1]), 0)
    onehot = (col == jnp.broadcast_to(bv, (NUM_GRAPHS, bv.shape[1]))
              ).astype(jnp.float32)
    pooled_acc[...] += jnp.dot(onehot, o, preferred_element_type=jnp.float32)

    @pl.when(i == pl.num_programs(0) - 1)
    def _():
        logits = jnp.dot(pooled_acc[...], wf_ref[...],
                         preferred_element_type=jnp.float32) + bf_ref[...]
        cols = lax.broadcasted_iota(jnp.int32, (NUM_GRAPHS, 128), 1)
        lm = jnp.where(cols < NUM_CLASSES, logits, -1e30)
        m = jnp.max(lm, axis=1, keepdims=True)
        ssum = jnp.sum(jnp.exp(lm - m), axis=1, keepdims=True)
        out_ref[...] = lm - m - jnp.log(ssum)


def _tc_pool(partials, sparts, b, batch_f, wf_pad, bf_pad):
    blk = 1024
    return pl.pallas_call(
        _tc_pool_body,
        grid=(NPAD // blk,),
        in_specs=[
            pl.BlockSpec((2, blk, HID), lambda i: (0, i, 0)),
            pl.BlockSpec((NW, blk), lambda i: (0, i)),
            pl.BlockSpec((1, HID), lambda i: (0, 0)),
            pl.BlockSpec((1, blk), lambda i: (0, i)),
            pl.BlockSpec((HID, 128), lambda i: (0, 0)),
            pl.BlockSpec((1, 128), lambda i: (0, 0)),
        ],
        out_specs=pl.BlockSpec((NUM_GRAPHS, 128), lambda i: (0, 0)),
        out_shape=jax.ShapeDtypeStruct((NUM_GRAPHS, 128), jnp.float32),
        scratch_shapes=[pltpu.VMEM((NUM_GRAPHS, HID), jnp.float32)],
    )(partials, sparts, b, batch_f, wf_pad, bf_pad)


_L2E = 1.4426950408889634
_LN2_HI = 0.693359375
_LN2_LO = -2.12194440e-4
_EXP_C = (1.0 / 720, 1.0 / 120, 1.0 / 24, 1.0 / 6, 0.5, 1.0, 1.0)


def _exp16(e):
    e = jnp.minimum(jnp.maximum(e, -87.0), 87.0)
    t = e * _L2E
    half = jnp.where(t < 0.0, -0.5, 0.5)
    k = (t + half).astype(jnp.int32)
    kf = k.astype(jnp.float32)
    u = (e - kf * _LN2_HI) - kf * _LN2_LO
    p = jnp.full((L,), _EXP_C[0], jnp.float32)
    for c in _EXP_C[1:]:
        p = p * u + c
    two_k = plsc.bitcast((k + 127) << 23, jnp.float32)
    return p * two_k


def _sc_edge_body(h_hbm, asrc_hbm, adst_hbm, src_hbm, dst_hbm,
                  out_hbm, sp_hbm,
                  asrc_v, adst_v, sden_v, srcb_v, dstb_v, exb_v, rows_v,
                  out_sh, sem):
    c = lax.axis_index("c")
    s = lax.axis_index("s")
    w = s * NC + c

    pltpu.sync_copy(asrc_hbm, asrc_v)
    pltpu.sync_copy(adst_hbm, adst_v)
    pltpu.sync_copy(src_hbm.at[w], srcb_v)
    pltpu.sync_copy(dst_hbm.at[w], dstb_v)

    zero16 = jnp.zeros((L,), jnp.float32)

    def _zero_sden(i, carry):
        sden_v[pl.ds(i * L, L)] = zero16
        return carry
    lax.fori_loop(0, NPAD // L, _zero_sden, 0)

    def _zero_rows(i, carry):
        r = i // 4
        q = i % 4
        rows_v[r, pl.ds(q * L, L)] = zero16
        return carry
    lax.fori_loop(0, KB * HID // L, _zero_rows, 0)
    base = s * ROWS_PER_TILE
    for k in range(ROWS_PER_TILE // KB):
        pltpu.sync_copy(rows_v, out_sh.at[pl.ds(base + k * KB, KB), :])
    plsc.subcore_barrier()

    def _block(j, carry):
        gather = pltpu.async_copy(h_hbm.at[srcb_v.at[j]], rows_v, sem)
        for jj in range(KB // L):
            sv = srcb_v[j, pl.ds(jj * L, L)]
            dv = dstb_v[j, pl.ds(jj * L, L)]
            ea = plsc.load_gather(asrc_v, [sv])
            eb = plsc.load_gather(adst_v, [dv])
            e = ea + eb
            e = jnp.where(e > 0.0, e, 0.2 * e)
            ex = _exp16(e)
            exb_v[pl.ds(jj * L, L)] = ex
            plsc.addupdate_scatter(sden_v, [dv], ex)
        gather.wait()
        for eidx in range(KB):
            av = plsc.load_gather(exb_v, [jnp.full((L,), eidx, jnp.int32)])
            for q in range(HID // L):
                rows_v[eidx, pl.ds(q * L, L)] = (
                    rows_v[eidx, pl.ds(q * L, L)] * av)
        pltpu.sync_copy(rows_v, out_sh.at[dstb_v.at[j]], add=True)
        return carry

    lax.fori_loop(0, BPW, _block, 0)
    plsc.subcore_barrier()

    pltpu.sync_copy(out_sh.at[pl.ds(base, ROWS_PER_TILE), :],
                    out_hbm.at[c, pl.ds(base, ROWS_PER_TILE), :])
    pltpu.sync_copy(sden_v, sp_hbm.at[w])


@functools.partial(
    pl.kernel,
    out_type=(
        jax.ShapeDtypeStruct((2, NPAD, HID), jnp.float32),
        jax.ShapeDtypeStruct((NW, NPAD), jnp.float32),
    ),
    mesh=plsc.VectorSubcoreMesh(
        core_axis_name="c", subcore_axis_name="s",
        num_cores=NC, num_subcores=NS),
    compiler_params=pltpu.CompilerParams(
        needs_layout_passes=False, use_tc_tiling_on_sc=False),
    scratch_types=[
        pltpu.VMEM((NPAD,), jnp.float32),
        pltpu.VMEM((NPAD,), jnp.float32),
        pltpu.VMEM((NPAD,), jnp.float32),
        pltpu.VMEM((BPW, KB), jnp.int32),
        pltpu.VMEM((BPW, KB), jnp.int32),
        pltpu.VMEM((KB,), jnp.float32),
        pltpu.VMEM((KB, HID), jnp.float32),
        pltpu.VMEM_SHARED((NPAD, HID), jnp.float32),
        pltpu.SemaphoreType.DMA,
    ],
)
def _sc_edge(h_hbm, asrc_hbm, adst_hbm, src_hbm, dst_hbm, out_hbm, sp_hbm,
             asrc_v, adst_v, sden_v, srcb_v, dstb_v, exb_v, rows_v,
             out_sh, sem):
    _sc_edge_body(h_hbm, asrc_hbm, adst_hbm, src_hbm, dst_hbm,
                  out_hbm, sp_hbm,
                  asrc_v, adst_v, sden_v, srcb_v, dstb_v, exb_v, rows_v,
                  out_sh, sem)


def _pack_attn(a_s, a_d):
    a = jnp.zeros((HID, 128), jnp.float32)
    a = a.at[:, 0].set(a_s)
    a = a.at[:, 1].set(a_d)
    return a


def kernel(x, edge_index, batch, W1, as1, ad1, b1, W2, as2, ad2, b2, Wf, bf):
    x_pad = jnp.zeros((NPAD, D_IN), jnp.float32).at[:N].set(x)
    loop = jnp.arange(N, dtype=jnp.int32)
    src = jnp.concatenate([edge_index[0].astype(jnp.int32), loop])
    dst = jnp.concatenate([edge_index[1].astype(jnp.int32), loop])
    src2d = jnp.full((E2PAD,), PAD_NODE, jnp.int32).at[:E2].set(src)
    dst2d = jnp.full((E2PAD,), PAD_NODE, jnp.int32).at[:E2].set(dst)
    src2d = src2d.reshape(NW, BPW, KB)
    dst2d = dst2d.reshape(NW, BPW, KB)
    batch_f = (jnp.full((NPAD,), NUM_GRAPHS, jnp.int32)
               .at[:N].set(batch.astype(jnp.int32))
               .reshape(1, NPAD))
    A1 = _pack_attn(as1, ad1)
    A2 = _pack_attn(as2, ad2)
    wf_pad = jnp.zeros((HID, 128), jnp.float32).at[:, :NUM_CLASSES].set(Wf)
    bf_pad = jnp.zeros((1, 128), jnp.float32).at[0, :NUM_CLASSES].set(bf)
    b1r = b1.reshape(1, HID)
    b2r = b2.reshape(1, HID)

    h1, aux1 = _tc_mm(x_pad, W1, A1)
    out1, sp1 = _sc_edge(h1, aux1[:, 0], aux1[:, 1], src2d, dst2d)
    h2, aux2 = _tc_norm_mm(out1, sp1, b1r, W2, A2)
    out2, sp2 = _sc_edge(h2, aux2[:, 0], aux2[:, 1], src2d, dst2d)
    out = _tc_pool(out2, sp2, b2r, batch_f, wf_pad, bf_pad)
    return out[:, :NUM_CLASSES]
